# Initial kernel scaffold; baseline (speedup 1.0000x reference)
import functools

import jax
import jax.numpy as jnp
from jax import lax
from jax.experimental import pallas as pl
from jax.experimental.pallas import tpu as pltpu
from jax.experimental.pallas import tpu_sc as plsc

N_NODES = 10000
N_EDGES = 320000
D = 128
NUM_RELS = 16
NUM_BASES = 8

NW = 32
EPW = N_EDGES // NW
CHUNK = 80
NCHUNK = EPW // CHUNK
RPT = N_NODES // 16


def _proj_body(wc_ref, w_ref, h_ref, out_ref):
    r = pl.program_id(0)
    rel_w = wc_ref[r, 0] * w_ref[0]
    for b in range(1, NUM_BASES):
        rel_w = rel_w + wc_ref[r, b] * w_ref[b]
    out_ref[0] = jnp.dot(h_ref[...], rel_w, preferred_element_type=jnp.float32)


def _proj(h, weight, w_comp):
    return pl.pallas_call(
        _proj_body,
        grid=(NUM_RELS,),
        in_specs=[
            pl.BlockSpec(memory_space=pltpu.SMEM),
            pl.BlockSpec((NUM_BASES, D, D), lambda r: (0, 0, 0)),
            pl.BlockSpec((N_NODES, D), lambda r: (0, 0)),
        ],
        out_specs=pl.BlockSpec((1, N_NODES, D), lambda r: (r, 0, 0)),
        out_shape=jax.ShapeDtypeStruct((NUM_RELS, N_NODES, D), jnp.float32),
    )(w_comp, weight, h)


def _sc_agg_body(proj_hbm, gidx_hbm, dst_hbm, zrow_hbm, zdeg_hbm,
                 part_out, deg_out,
                 gidx_v, dst_v, rows_v, ones_v, acc_sh, deg_sh, sem):
    cid = lax.axis_index("c")
    sid = lax.axis_index("s")
    wid = sid * 2 + cid

    pltpu.sync_copy(zrow_hbm, acc_sh.at[pl.ds(sid * RPT, RPT)])
    pltpu.sync_copy(zdeg_hbm, deg_sh.at[pl.ds(sid * RPT, RPT)])

    def fill_ones(i, carry):
        ones_v[i] = jnp.ones((16,), jnp.float32)
        return carry

    lax.fori_loop(0, CHUNK, fill_ones, 0)
    plsc.subcore_barrier()

    base0 = wid * EPW

    def body(i, carry):
        base = base0 + i * CHUNK
        pltpu.sync_copy(gidx_hbm.at[pl.ds(base, CHUNK)], gidx_v)
        pltpu.sync_copy(dst_hbm.at[pl.ds(base, CHUNK)], dst_v)
        pltpu.async_copy(proj_hbm.at[gidx_v], rows_v, sem).wait()
        pltpu.sync_copy(rows_v, acc_sh.at[dst_v], add=True)
        pltpu.sync_copy(ones_v, deg_sh.at[dst_v], add=True)
        return carry

    lax.fori_loop(0, NCHUNK, body, 0)
    plsc.subcore_barrier()

    pltpu.sync_copy(acc_sh.at[pl.ds(sid * RPT, RPT)],
                    part_out.at[cid, pl.ds(sid * RPT, RPT)])
    pltpu.sync_copy(deg_sh.at[pl.ds(sid * RPT, RPT)],
                    deg_out.at[cid, pl.ds(sid * RPT, RPT)])


_sc_agg = functools.partial(
    pl.kernel,
    mesh=plsc.VectorSubcoreMesh(core_axis_name="c", subcore_axis_name="s"),
    out_type=[
        jax.ShapeDtypeStruct((2, N_NODES, D), jnp.float32),
        jax.ShapeDtypeStruct((2, N_NODES, 16), jnp.float32),
    ],
    scratch_types=[
        pltpu.VMEM((CHUNK,), jnp.int32),
        pltpu.VMEM((CHUNK,), jnp.int32),
        pltpu.VMEM((CHUNK, D), jnp.float32),
        pltpu.VMEM((CHUNK, 16), jnp.float32),
        pltpu.VMEM_SHARED((N_NODES, D), jnp.float32),
        pltpu.VMEM_SHARED((N_NODES, 16), jnp.float32),
        pltpu.SemaphoreType.DMA,
    ],
)(_sc_agg_body)


def _combine_body(part_ref, degp_ref, h_ref, slw_ref, out_ref):
    deg = jnp.sum(degp_ref[0] + degp_ref[1], axis=1, keepdims=True) * (1.0 / 16.0)
    s = jnp.log(deg + 1.0)
    mean = jnp.sum(s) * (1.0 / N_NODES)
    scale = s / mean
    nei = part_ref[0] + part_ref[1]
    out_ref[...] = (
        jnp.dot(h_ref[...], slw_ref[...], preferred_element_type=jnp.float32)
        + nei * scale
    )


def _combine(part, degp, h, slw):
    return pl.pallas_call(
        _combine_body,
        out_shape=jax.ShapeDtypeStruct((N_NODES, D), jnp.float32),
    )(part, degp, h, slw)


def kernel(h, edge_index, edge_type, weight, w_comp, self_loop_weight):
    src = edge_index[0].astype(jnp.int32)
    dst = edge_index[1].astype(jnp.int32)
    gidx = edge_type.astype(jnp.int32) * N_NODES + src

    all_proj = _proj(h, weight, w_comp).reshape(NUM_RELS * N_NODES, D)

    zrow = jnp.zeros((RPT, D), jnp.float32)
    zdeg = jnp.zeros((RPT, 16), jnp.float32)
    part, degp = _sc_agg(all_proj, gidx, dst, zrow, zdeg)

    return _combine(part, degp, h, self_loop_weight)


# speedup vs baseline: 2.9433x; 2.9433x over previous
import functools

import jax
import jax.numpy as jnp
from jax import lax
from jax.experimental import pallas as pl
from jax.experimental.pallas import tpu as pltpu
from jax.experimental.pallas import tpu_sc as plsc

N_NODES = 10000
N_EDGES = 320000
D = 128
NUM_RELS = 16
NUM_BASES = 8

NW = 32
EPW = N_EDGES // NW
CHUNK = 80
NCHUNK = EPW // CHUNK
N_PAD = 10240
RPT = N_PAD // 16


def _proj_body(wc_ref, w_ref, h_ref, out_ref):
    r = pl.program_id(0)
    rel_w = wc_ref[r, 0] * w_ref[0]
    for b in range(1, NUM_BASES):
        rel_w = rel_w + wc_ref[r, b] * w_ref[b]
    out_ref[0] = jnp.dot(h_ref[...], rel_w, preferred_element_type=jnp.float32)


def _proj(h, weight, w_comp):
    return pl.pallas_call(
        _proj_body,
        grid=(NUM_RELS,),
        in_specs=[
            pl.BlockSpec(memory_space=pltpu.SMEM),
            pl.BlockSpec((NUM_BASES, D, D), lambda r: (0, 0, 0)),
            pl.BlockSpec((N_NODES, D), lambda r: (0, 0)),
        ],
        out_specs=pl.BlockSpec((1, N_NODES, D), lambda r: (r, 0, 0)),
        out_shape=jax.ShapeDtypeStruct((NUM_RELS, N_NODES, D), jnp.float32),
    )(w_comp, weight, h)


def _sc_agg_body(proj_hbm, gidx_hbm, dst_hbm, zrow_hbm,
                 part_out, deg_out,
                 gidx_v, dst_v, rows_v, ones_v, acc_sh, sem):
    cid = lax.axis_index("c")
    sid = lax.axis_index("s")
    wid = sid * 2 + cid

    pltpu.sync_copy(zrow_hbm, acc_sh.at[pl.ds(sid * RPT, RPT)])

    def fill_ones(i, carry):
        for j in range(D // 16):
            ones_v[i, pl.ds(j * 16, 16)] = jnp.ones((16,), jnp.float32)
        return carry

    lax.fori_loop(0, CHUNK, fill_ones, 0)
    plsc.subcore_barrier()

    base0 = wid * EPW

    def body(i, carry):
        base = base0 + i * CHUNK
        pltpu.sync_copy(gidx_hbm.at[pl.ds(base, CHUNK)], gidx_v)
        pltpu.sync_copy(dst_hbm.at[pl.ds(base, CHUNK)], dst_v)
        pltpu.async_copy(proj_hbm.at[gidx_v], rows_v, sem).wait()
        pltpu.sync_copy(rows_v, acc_sh.at[dst_v], add=True)
        return carry

    lax.fori_loop(0, NCHUNK, body, 0)
    plsc.subcore_barrier()

    pltpu.sync_copy(acc_sh.at[pl.ds(sid * RPT, RPT)],
                    part_out.at[cid, pl.ds(sid * RPT, RPT)])
    plsc.subcore_barrier()

    pltpu.sync_copy(zrow_hbm, acc_sh.at[pl.ds(sid * RPT, RPT)])
    plsc.subcore_barrier()

    def body2(i, carry):
        base = base0 + i * CHUNK
        pltpu.sync_copy(dst_hbm.at[pl.ds(base, CHUNK)], dst_v)
        pltpu.sync_copy(ones_v, acc_sh.at[dst_v], add=True)
        return carry

    lax.fori_loop(0, NCHUNK, body2, 0)
    plsc.subcore_barrier()

    pltpu.sync_copy(acc_sh.at[pl.ds(sid * RPT, RPT)],
                    deg_out.at[cid, pl.ds(sid * RPT, RPT)])


_sc_agg = functools.partial(
    pl.kernel,
    mesh=plsc.VectorSubcoreMesh(core_axis_name="c", subcore_axis_name="s"),
    out_type=[
        jax.ShapeDtypeStruct((2, N_PAD, D), jnp.float32),
        jax.ShapeDtypeStruct((2, N_PAD, D), jnp.float32),
    ],
    scratch_types=[
        pltpu.VMEM((CHUNK,), jnp.int32),
        pltpu.VMEM((CHUNK,), jnp.int32),
        pltpu.VMEM((CHUNK, D), jnp.float32),
        pltpu.VMEM((CHUNK, D), jnp.float32),
        pltpu.VMEM_SHARED((N_PAD, D), jnp.float32),
        pltpu.SemaphoreType.DMA,
    ],
)(_sc_agg_body)


def _combine_body(part_ref, degp_ref, h_ref, slw_ref, out_ref):
    deg = degp_ref[0, :N_NODES] + degp_ref[1, :N_NODES]
    s = jnp.log(deg + 1.0)
    mean = jnp.sum(s) * (1.0 / (N_NODES * D))
    scale = s * (1.0 / mean)
    nei = part_ref[0, :N_NODES] + part_ref[1, :N_NODES]
    out_ref[...] = (
        jnp.dot(h_ref[...], slw_ref[...], preferred_element_type=jnp.float32)
        + nei * scale
    )


def _combine(part, degp, h, slw):
    return pl.pallas_call(
        _combine_body,
        out_shape=jax.ShapeDtypeStruct((N_NODES, D), jnp.float32),
    )(part, degp, h, slw)


def kernel(h, edge_index, edge_type, weight, w_comp, self_loop_weight):
    src = edge_index[0].astype(jnp.int32)
    dst = edge_index[1].astype(jnp.int32)
    gidx = edge_type.astype(jnp.int32) * N_NODES + src

    all_proj = _proj(h, weight, w_comp).reshape(NUM_RELS * N_NODES, D)

    zrow = jnp.zeros((RPT, D), jnp.float32)
    part, degp = _sc_agg(all_proj, gidx, dst, zrow)

    return _combine(part, degp, h, self_loop_weight)
